# initial kernel scaffold (unmeasured)
import jax
import jax.numpy as jnp
from jax import lax
from jax.experimental import pallas as pl
from jax.experimental.pallas import tpu as pltpu

N_DEV = 8


def kernel(x, w_mat, scale_x, scale_w):
    m, k_sh = x.shape
    n = w_mat.shape[1]
    ch = m // N_DEV

    def body(x_ref, w_ref, sx_ref, sw_ref, out_ref,
             comm_ref, send_sem, recv_sem, ag_send_sem, ag_recv_sems,
             credit_sem):
        my = lax.axis_index("i")
        left = lax.rem(my + N_DEV - 1, N_DEV)
        right = lax.rem(my + 1, N_DEV)

        barrier_sem = pltpu.get_barrier_semaphore()
        for nbr in (left, right):
            pl.semaphore_signal(barrier_sem, inc=1, device_id=(nbr,),
                                device_id_type=pl.DeviceIdType.MESH)
        pl.semaphore_wait(barrier_sem, 2)

        xb = x_ref[...].astype(jnp.bfloat16)
        wb = w_ref[...].astype(jnp.bfloat16)
        out_ref[...] = jnp.dot(xb, wb, preferred_element_type=jnp.float32)

        def rows(c):
            return pl.ds(c * ch, ch)

        for s in range(N_DEV - 1):
            c_send = lax.rem(my - s + N_DEV, N_DEV)
            c_recv = lax.rem(my - s - 1 + N_DEV, N_DEV)
            if s > 0:
                pl.semaphore_wait(credit_sem, 1)
            rdma = pltpu.make_async_remote_copy(
                src_ref=out_ref.at[rows(c_send), :],
                dst_ref=comm_ref,
                send_sem=send_sem,
                recv_sem=recv_sem,
                device_id=(right,),
                device_id_type=pl.DeviceIdType.MESH,
            )
            rdma.start()
            rdma.wait()
            out_ref[rows(c_recv), :] += comm_ref[...]
            pl.semaphore_signal(credit_sem, inc=1, device_id=(left,),
                                device_id_type=pl.DeviceIdType.MESH)

        c_star = lax.rem(my + 1, N_DEV)
        scale = sx_ref[0] * sw_ref[0]
        out_ref[rows(c_star), :] = jnp.maximum(
            out_ref[rows(c_star), :] * scale, 0.0)

        for s in range(N_DEV - 1):
            c_send = lax.rem(my + 1 - s + N_DEV, N_DEV)
            rdma = pltpu.make_async_remote_copy(
                src_ref=out_ref.at[rows(c_send), :],
                dst_ref=out_ref.at[rows(c_send), :],
                send_sem=ag_send_sem,
                recv_sem=ag_recv_sems.at[s],
                device_id=(right,),
                device_id_type=pl.DeviceIdType.MESH,
            )
            rdma.start()
            rdma.wait()

    return pl.pallas_call(
        body,
        out_shape=jax.ShapeDtypeStruct((m, n), jnp.float32),
        in_specs=[
            pl.BlockSpec(memory_space=pltpu.VMEM),
            pl.BlockSpec(memory_space=pltpu.VMEM),
            pl.BlockSpec(memory_space=pltpu.SMEM),
            pl.BlockSpec(memory_space=pltpu.SMEM),
        ],
        out_specs=pl.BlockSpec(memory_space=pltpu.VMEM),
        scratch_shapes=[
            pltpu.VMEM((ch, n), jnp.float32),
            pltpu.SemaphoreType.DMA,
            pltpu.SemaphoreType.DMA,
            pltpu.SemaphoreType.DMA,
            pltpu.SemaphoreType.DMA((N_DEV - 1,)),
            pltpu.SemaphoreType.REGULAR,
        ],
        compiler_params=pltpu.CompilerParams(collective_id=0),
    )(x, w_mat, scale_x, scale_w)


# baseline (device time: 718017 ns/iter reference)
import jax
import jax.numpy as jnp
from jax import lax
from jax.experimental import pallas as pl
from jax.experimental.pallas import tpu as pltpu

N_DEV = 8


def kernel(x, w_mat, scale_x, scale_w):
    m, k_sh = x.shape
    n = w_mat.shape[1]
    ch = m // N_DEV

    def body(x_ref, w_ref, sx_ref, sw_ref, out_ref,
             comm_ref, send_sem, recv_sem, ag_send_sem, ag_recv_sems,
             credit_sem):
        my = lax.axis_index("i")
        left = lax.rem(my + N_DEV - 1, N_DEV)
        right = lax.rem(my + 1, N_DEV)

        barrier_sem = pltpu.get_barrier_semaphore()
        for nbr in (left, right):
            pl.semaphore_signal(barrier_sem, inc=1, device_id=(nbr,),
                                device_id_type=pl.DeviceIdType.MESH)
        pl.semaphore_wait(barrier_sem, 2)

        xb = x_ref[...].astype(jnp.bfloat16)
        wb = w_ref[...].astype(jnp.bfloat16)
        out_ref[...] = jnp.dot(xb, wb, preferred_element_type=jnp.float32)

        def rows(c):
            return pl.ds(c * ch, ch)

        for s in range(N_DEV - 1):
            c_send = lax.rem(my - s + N_DEV, N_DEV)
            c_recv = lax.rem(my - s - 1 + N_DEV, N_DEV)
            if s > 0:
                pl.semaphore_wait(credit_sem, 1)
            rdma = pltpu.make_async_remote_copy(
                src_ref=out_ref.at[rows(c_send), :],
                dst_ref=comm_ref,
                send_sem=send_sem,
                recv_sem=recv_sem,
                device_id=(right,),
                device_id_type=pl.DeviceIdType.MESH,
            )
            rdma.start()
            rdma.wait()
            out_ref[rows(c_recv), :] += comm_ref[...]
            pl.semaphore_signal(credit_sem, inc=1, device_id=(left,),
                                device_id_type=pl.DeviceIdType.MESH)
        pl.semaphore_wait(credit_sem, 1)

        c_star = lax.rem(my + 1, N_DEV)
        scale = sx_ref[0] * sw_ref[0]
        out_ref[rows(c_star), :] = jnp.maximum(
            out_ref[rows(c_star), :] * scale, 0.0)

        for s in range(N_DEV - 1):
            c_send = lax.rem(my + 1 - s + N_DEV, N_DEV)
            rdma = pltpu.make_async_remote_copy(
                src_ref=out_ref.at[rows(c_send), :],
                dst_ref=out_ref.at[rows(c_send), :],
                send_sem=ag_send_sem,
                recv_sem=ag_recv_sems.at[s],
                device_id=(right,),
                device_id_type=pl.DeviceIdType.MESH,
            )
            rdma.start()
            rdma.wait()

    return pl.pallas_call(
        body,
        out_shape=jax.ShapeDtypeStruct((m, n), jnp.float32),
        in_specs=[
            pl.BlockSpec(memory_space=pltpu.VMEM),
            pl.BlockSpec(memory_space=pltpu.VMEM),
            pl.BlockSpec(memory_space=pltpu.SMEM),
            pl.BlockSpec(memory_space=pltpu.SMEM),
        ],
        out_specs=pl.BlockSpec(memory_space=pltpu.VMEM),
        scratch_shapes=[
            pltpu.VMEM((ch, n), jnp.float32),
            pltpu.SemaphoreType.DMA,
            pltpu.SemaphoreType.DMA,
            pltpu.SemaphoreType.DMA,
            pltpu.SemaphoreType.DMA((N_DEV - 1,)),
            pltpu.SemaphoreType.REGULAR,
        ],
        compiler_params=pltpu.CompilerParams(
            collective_id=0,
            vmem_limit_bytes=60 * 1024 * 1024,
        ),
    )(x, w_mat, scale_x, scale_w)


# device time: 408570 ns/iter; 1.7574x vs baseline; 1.7574x over previous
import jax
import jax.numpy as jnp
from jax import lax
from jax.experimental import pallas as pl
from jax.experimental.pallas import tpu as pltpu

N_DEV = 8


def kernel(x, w_mat, scale_x, scale_w):
    m, k_sh = x.shape
    n = w_mat.shape[1]
    ch = m // N_DEV
    hf = ch // 2

    def body(x_ref, w_ref, sx_ref, sw_ref, out_ref,
             comm_f, comm_b,
             send_f, recv_f, send_b, recv_b,
             ag_send_f, ag_recv_f, ag_send_b, ag_recv_b,
             credit_f, credit_b):
        my = lax.axis_index("i")
        left = lax.rem(my + N_DEV - 1, N_DEV)
        right = lax.rem(my + 1, N_DEV)

        barrier_sem = pltpu.get_barrier_semaphore()
        for nbr in (left, right):
            pl.semaphore_signal(barrier_sem, inc=1, device_id=(nbr,),
                                device_id_type=pl.DeviceIdType.MESH)
        pl.semaphore_wait(barrier_sem, 2)

        xb = x_ref[...].astype(jnp.bfloat16)
        wb = w_ref[...].astype(jnp.bfloat16)
        out_ref[...] = jnp.dot(xb, wb, preferred_element_type=jnp.float32)

        def frows(c):
            return pl.ds(c * ch, hf)

        def brows(c):
            return pl.ds(c * ch + hf, hf)

        for s in range(N_DEV - 1):
            cs_f = lax.rem(my - s + N_DEV, N_DEV)
            cr_f = lax.rem(my - s - 1 + N_DEV, N_DEV)
            cs_b = lax.rem(my + s, N_DEV)
            cr_b = lax.rem(my + s + 1, N_DEV)
            if s > 0:
                pl.semaphore_wait(credit_f, 1)
                pl.semaphore_wait(credit_b, 1)
            rdma_f = pltpu.make_async_remote_copy(
                src_ref=out_ref.at[frows(cs_f), :],
                dst_ref=comm_f,
                send_sem=send_f,
                recv_sem=recv_f,
                device_id=(right,),
                device_id_type=pl.DeviceIdType.MESH,
            )
            rdma_b = pltpu.make_async_remote_copy(
                src_ref=out_ref.at[brows(cs_b), :],
                dst_ref=comm_b,
                send_sem=send_b,
                recv_sem=recv_b,
                device_id=(left,),
                device_id_type=pl.DeviceIdType.MESH,
            )
            rdma_f.start()
            rdma_b.start()
            rdma_f.wait()
            rdma_b.wait()
            out_ref[frows(cr_f), :] += comm_f[...]
            out_ref[brows(cr_b), :] += comm_b[...]
            pl.semaphore_signal(credit_f, inc=1, device_id=(left,),
                                device_id_type=pl.DeviceIdType.MESH)
            pl.semaphore_signal(credit_b, inc=1, device_id=(right,),
                                device_id_type=pl.DeviceIdType.MESH)
        pl.semaphore_wait(credit_f, 1)
        pl.semaphore_wait(credit_b, 1)

        scale = sx_ref[0] * sw_ref[0]
        c_star_f = lax.rem(my + 1, N_DEV)
        c_star_b = left
        out_ref[frows(c_star_f), :] = jnp.maximum(
            out_ref[frows(c_star_f), :] * scale, 0.0)
        out_ref[brows(c_star_b), :] = jnp.maximum(
            out_ref[brows(c_star_b), :] * scale, 0.0)

        for s in range(N_DEV - 1):
            c_f = lax.rem(my + 1 - s + N_DEV, N_DEV)
            c_b = lax.rem(my - 1 + s + N_DEV, N_DEV)
            rdma_f = pltpu.make_async_remote_copy(
                src_ref=out_ref.at[frows(c_f), :],
                dst_ref=out_ref.at[frows(c_f), :],
                send_sem=ag_send_f,
                recv_sem=ag_recv_f.at[s],
                device_id=(right,),
                device_id_type=pl.DeviceIdType.MESH,
            )
            rdma_b = pltpu.make_async_remote_copy(
                src_ref=out_ref.at[brows(c_b), :],
                dst_ref=out_ref.at[brows(c_b), :],
                send_sem=ag_send_b,
                recv_sem=ag_recv_b.at[s],
                device_id=(left,),
                device_id_type=pl.DeviceIdType.MESH,
            )
            rdma_f.start()
            rdma_b.start()
            rdma_f.wait()
            rdma_b.wait()

    return pl.pallas_call(
        body,
        out_shape=jax.ShapeDtypeStruct((m, n), jnp.float32),
        in_specs=[
            pl.BlockSpec(memory_space=pltpu.VMEM),
            pl.BlockSpec(memory_space=pltpu.VMEM),
            pl.BlockSpec(memory_space=pltpu.SMEM),
            pl.BlockSpec(memory_space=pltpu.SMEM),
        ],
        out_specs=pl.BlockSpec(memory_space=pltpu.VMEM),
        scratch_shapes=[
            pltpu.VMEM((hf, n), jnp.float32),
            pltpu.VMEM((hf, n), jnp.float32),
            pltpu.SemaphoreType.DMA,
            pltpu.SemaphoreType.DMA,
            pltpu.SemaphoreType.DMA,
            pltpu.SemaphoreType.DMA,
            pltpu.SemaphoreType.DMA,
            pltpu.SemaphoreType.DMA((N_DEV - 1,)),
            pltpu.SemaphoreType.DMA,
            pltpu.SemaphoreType.DMA((N_DEV - 1,)),
            pltpu.SemaphoreType.REGULAR,
            pltpu.SemaphoreType.REGULAR,
        ],
        compiler_params=pltpu.CompilerParams(
            collective_id=0,
            vmem_limit_bytes=60 * 1024 * 1024,
        ),
    )(x, w_mat, scale_x, scale_w)


# device time: 262246 ns/iter; 2.7380x vs baseline; 1.5580x over previous
import jax
import jax.numpy as jnp
from jax import lax
from jax.experimental import pallas as pl
from jax.experimental.pallas import tpu as pltpu

N_DEV = 8


def kernel(x, w_mat, scale_x, scale_w):
    m, k_sh = x.shape
    n = w_mat.shape[1]
    ch = m // N_DEV
    hf = ch // 2

    def body(x_ref, w_ref, sx_ref, sw_ref, out_ref,
             gather_ref, comm_f, comm_b, stage_f, stage_b,
             send_f, recv_f, send_b, recv_b,
             ag_send_f, ag_recv_f, ag_send_b, ag_recv_b,
             credit_f, credit_b):
        my = lax.axis_index("i")
        left = lax.rem(my + N_DEV - 1, N_DEV)
        right = lax.rem(my + 1, N_DEV)

        barrier_sem = pltpu.get_barrier_semaphore()
        for nbr in (left, right):
            pl.semaphore_signal(barrier_sem, inc=1, device_id=(nbr,),
                                device_id_type=pl.DeviceIdType.MESH)
        pl.semaphore_wait(barrier_sem, 2)

        for c in range(N_DEV):
            out_ref[pl.ds(c * ch, ch), :] = jnp.dot(
                x_ref[pl.ds(c * ch, ch), :], w_ref[...],
                preferred_element_type=jnp.float32)

        def frows(c):
            return pl.ds(c * ch, hf)

        def brows(c):
            return pl.ds(c * ch + hf, hf)

        for s in range(N_DEV - 1):
            cs_f = lax.rem(my - s + N_DEV, N_DEV)
            cr_f = lax.rem(my - s - 1 + N_DEV, N_DEV)
            cs_b = lax.rem(my + s, N_DEV)
            cr_b = lax.rem(my + s + 1, N_DEV)
            stage_f[...] = out_ref[frows(cs_f), :].astype(jnp.bfloat16)
            stage_b[...] = out_ref[brows(cs_b), :].astype(jnp.bfloat16)
            if s > 0:
                pl.semaphore_wait(credit_f, 1)
                pl.semaphore_wait(credit_b, 1)
            rdma_f = pltpu.make_async_remote_copy(
                src_ref=stage_f,
                dst_ref=comm_f,
                send_sem=send_f,
                recv_sem=recv_f,
                device_id=(right,),
                device_id_type=pl.DeviceIdType.MESH,
            )
            rdma_b = pltpu.make_async_remote_copy(
                src_ref=stage_b,
                dst_ref=comm_b,
                send_sem=send_b,
                recv_sem=recv_b,
                device_id=(left,),
                device_id_type=pl.DeviceIdType.MESH,
            )
            rdma_f.start()
            rdma_b.start()
            rdma_f.wait()
            rdma_b.wait()
            out_ref[frows(cr_f), :] += comm_f[...].astype(jnp.float32)
            out_ref[brows(cr_b), :] += comm_b[...].astype(jnp.float32)
            pl.semaphore_signal(credit_f, inc=1, device_id=(left,),
                                device_id_type=pl.DeviceIdType.MESH)
            pl.semaphore_signal(credit_b, inc=1, device_id=(right,),
                                device_id_type=pl.DeviceIdType.MESH)
        pl.semaphore_wait(credit_f, 1)
        pl.semaphore_wait(credit_b, 1)

        c_star_f = lax.rem(my + 1, N_DEV)
        c_star_b = left
        gather_ref[frows(c_star_f), :] = (
            out_ref[frows(c_star_f), :].astype(jnp.bfloat16))
        gather_ref[brows(c_star_b), :] = (
            out_ref[brows(c_star_b), :].astype(jnp.bfloat16))

        for s in range(N_DEV - 1):
            c_f = lax.rem(my + 1 - s + N_DEV, N_DEV)
            c_b = lax.rem(my - 1 + s + N_DEV, N_DEV)
            rdma_f = pltpu.make_async_remote_copy(
                src_ref=gather_ref.at[frows(c_f), :],
                dst_ref=gather_ref.at[frows(c_f), :],
                send_sem=ag_send_f,
                recv_sem=ag_recv_f.at[s],
                device_id=(right,),
                device_id_type=pl.DeviceIdType.MESH,
            )
            rdma_b = pltpu.make_async_remote_copy(
                src_ref=gather_ref.at[brows(c_b), :],
                dst_ref=gather_ref.at[brows(c_b), :],
                send_sem=ag_send_b,
                recv_sem=ag_recv_b.at[s],
                device_id=(left,),
                device_id_type=pl.DeviceIdType.MESH,
            )
            rdma_f.start()
            rdma_b.start()
            rdma_f.wait()
            rdma_b.wait()

        scale = sx_ref[0] * sw_ref[0]
        for c in range(N_DEV):
            out_ref[pl.ds(c * ch, ch), :] = jnp.maximum(
                gather_ref[pl.ds(c * ch, ch), :].astype(jnp.float32)
                * scale, 0.0)

    out = pl.pallas_call(
        body,
        out_shape=jax.ShapeDtypeStruct((m, n), jnp.float32),
        in_specs=[
            pl.BlockSpec(memory_space=pltpu.VMEM),
            pl.BlockSpec(memory_space=pltpu.VMEM),
            pl.BlockSpec(memory_space=pltpu.SMEM),
            pl.BlockSpec(memory_space=pltpu.SMEM),
        ],
        out_specs=pl.BlockSpec(memory_space=pltpu.VMEM),
        scratch_shapes=[
            pltpu.VMEM((m, n), jnp.bfloat16),
            pltpu.VMEM((hf, n), jnp.bfloat16),
            pltpu.VMEM((hf, n), jnp.bfloat16),
            pltpu.VMEM((hf, n), jnp.bfloat16),
            pltpu.VMEM((hf, n), jnp.bfloat16),
            pltpu.SemaphoreType.DMA,
            pltpu.SemaphoreType.DMA,
            pltpu.SemaphoreType.DMA,
            pltpu.SemaphoreType.DMA,
            pltpu.SemaphoreType.DMA,
            pltpu.SemaphoreType.DMA((N_DEV - 1,)),
            pltpu.SemaphoreType.DMA,
            pltpu.SemaphoreType.DMA((N_DEV - 1,)),
            pltpu.SemaphoreType.REGULAR,
            pltpu.SemaphoreType.REGULAR,
        ],
        compiler_params=pltpu.CompilerParams(
            collective_id=0,
            vmem_limit_bytes=62 * 1024 * 1024,
        ),
    )(x.astype(jnp.bfloat16), w_mat.astype(jnp.bfloat16), scale_x, scale_w)
    return out


# device time: 253465 ns/iter; 2.8328x vs baseline; 1.0346x over previous
import jax
import jax.numpy as jnp
from jax import lax
from jax.experimental import pallas as pl
from jax.experimental.pallas import tpu as pltpu

N_DEV = 8


def kernel(x, w_mat, scale_x, scale_w):
    m, k_sh = x.shape
    n = w_mat.shape[1]
    ch = m // N_DEV
    hf = ch // 2

    def body(x_ref, w_ref, sx_ref, sw_ref, out_ref,
             gather_ref, comm_f, comm_b, stage_f, stage_b,
             send_f, recv_f, send_b, recv_b,
             ag_send_f, ag_recv_f, ag_send_b, ag_recv_b,
             credit_f, credit_b):
        my = lax.axis_index("i")
        left = lax.rem(my + N_DEV - 1, N_DEV)
        right = lax.rem(my + 1, N_DEV)

        barrier_sem = pltpu.get_barrier_semaphore()
        for nbr in (left, right):
            pl.semaphore_signal(barrier_sem, inc=1, device_id=(nbr,),
                                device_id_type=pl.DeviceIdType.MESH)
        pl.semaphore_wait(barrier_sem, 2)

        def frows(c):
            return pl.ds(c * ch, hf)

        def brows(c):
            return pl.ds(c * ch + hf, hf)

        def gemm_chunk(c):
            out_ref[pl.ds(c * ch, ch), :] = jnp.dot(
                x_ref[pl.ds(c * ch, ch), :], w_ref[...],
                preferred_element_type=jnp.float32)

        gemm_chunk(my)
        stage_f[...] = out_ref[frows(my), :].astype(jnp.bfloat16)
        stage_b[...] = out_ref[brows(my), :].astype(jnp.bfloat16)

        c_star_f = right
        c_star_b = left

        for s in range(N_DEV - 1):
            cr_f = lax.rem(my - s - 1 + N_DEV, N_DEV)
            cr_b = lax.rem(my + s + 1, N_DEV)
            if s > 0:
                pl.semaphore_wait(credit_f, 1)
                pl.semaphore_wait(credit_b, 1)
            rdma_f = pltpu.make_async_remote_copy(
                src_ref=stage_f,
                dst_ref=comm_f,
                send_sem=send_f,
                recv_sem=recv_f,
                device_id=(right,),
                device_id_type=pl.DeviceIdType.MESH,
            )
            rdma_b = pltpu.make_async_remote_copy(
                src_ref=stage_b,
                dst_ref=comm_b,
                send_sem=send_b,
                recv_sem=recv_b,
                device_id=(left,),
                device_id_type=pl.DeviceIdType.MESH,
            )
            rdma_f.start()
            rdma_b.start()
            if s == 0:
                for j in range(1, N_DEV):
                    gemm_chunk(lax.rem(my + j, N_DEV))
            rdma_f.wait()
            rdma_b.wait()
            acc_f = out_ref[frows(cr_f), :] + comm_f[...].astype(jnp.float32)
            acc_b = out_ref[brows(cr_b), :] + comm_b[...].astype(jnp.float32)
            out_ref[frows(cr_f), :] = acc_f
            out_ref[brows(cr_b), :] = acc_b
            if s < N_DEV - 2:
                stage_f[...] = acc_f.astype(jnp.bfloat16)
                stage_b[...] = acc_b.astype(jnp.bfloat16)
            else:
                gather_ref[frows(c_star_f), :] = acc_f.astype(jnp.bfloat16)
                gather_ref[brows(c_star_b), :] = acc_b.astype(jnp.bfloat16)
            pl.semaphore_signal(credit_f, inc=1, device_id=(left,),
                                device_id_type=pl.DeviceIdType.MESH)
            pl.semaphore_signal(credit_b, inc=1, device_id=(right,),
                                device_id_type=pl.DeviceIdType.MESH)
        pl.semaphore_wait(credit_f, 1)
        pl.semaphore_wait(credit_b, 1)

        scale = sx_ref[0] * sw_ref[0]

        def epilogue_half(rows_slice):
            out_ref[rows_slice, :] = jnp.maximum(
                gather_ref[rows_slice, :].astype(jnp.float32) * scale, 0.0)

        for s in range(N_DEV - 1):
            c_f = lax.rem(my + 1 - s + N_DEV, N_DEV)
            c_b = lax.rem(my - 1 + s + N_DEV, N_DEV)
            rdma_f = pltpu.make_async_remote_copy(
                src_ref=gather_ref.at[frows(c_f), :],
                dst_ref=gather_ref.at[frows(c_f), :],
                send_sem=ag_send_f,
                recv_sem=ag_recv_f.at[s],
                device_id=(right,),
                device_id_type=pl.DeviceIdType.MESH,
            )
            rdma_b = pltpu.make_async_remote_copy(
                src_ref=gather_ref.at[brows(c_b), :],
                dst_ref=gather_ref.at[brows(c_b), :],
                send_sem=ag_send_b,
                recv_sem=ag_recv_b.at[s],
                device_id=(left,),
                device_id_type=pl.DeviceIdType.MESH,
            )
            rdma_f.start()
            rdma_b.start()
            epilogue_half(frows(c_f))
            epilogue_half(brows(c_b))
            rdma_f.wait()
            rdma_b.wait()
        epilogue_half(frows(lax.rem(my + 2, N_DEV)))
        epilogue_half(brows(lax.rem(my - 2 + N_DEV, N_DEV)))

    out = pl.pallas_call(
        body,
        out_shape=jax.ShapeDtypeStruct((m, n), jnp.float32),
        in_specs=[
            pl.BlockSpec(memory_space=pltpu.VMEM),
            pl.BlockSpec(memory_space=pltpu.VMEM),
            pl.BlockSpec(memory_space=pltpu.SMEM),
            pl.BlockSpec(memory_space=pltpu.SMEM),
        ],
        out_specs=pl.BlockSpec(memory_space=pltpu.VMEM),
        scratch_shapes=[
            pltpu.VMEM((m, n), jnp.bfloat16),
            pltpu.VMEM((hf, n), jnp.bfloat16),
            pltpu.VMEM((hf, n), jnp.bfloat16),
            pltpu.VMEM((hf, n), jnp.bfloat16),
            pltpu.VMEM((hf, n), jnp.bfloat16),
            pltpu.SemaphoreType.DMA,
            pltpu.SemaphoreType.DMA,
            pltpu.SemaphoreType.DMA,
            pltpu.SemaphoreType.DMA,
            pltpu.SemaphoreType.DMA,
            pltpu.SemaphoreType.DMA((N_DEV - 1,)),
            pltpu.SemaphoreType.DMA,
            pltpu.SemaphoreType.DMA((N_DEV - 1,)),
            pltpu.SemaphoreType.REGULAR,
            pltpu.SemaphoreType.REGULAR,
        ],
        compiler_params=pltpu.CompilerParams(
            collective_id=0,
            vmem_limit_bytes=62 * 1024 * 1024,
        ),
    )(x.astype(jnp.bfloat16), w_mat.astype(jnp.bfloat16), scale_x, scale_w)
    return out


# device time: 250683 ns/iter; 2.8642x vs baseline; 1.0111x over previous
import jax
import jax.numpy as jnp
from jax import lax
from jax.experimental import pallas as pl
from jax.experimental.pallas import tpu as pltpu

N_DEV = 8


def kernel(x, w_mat, scale_x, scale_w):
    m, k_sh = x.shape
    n = w_mat.shape[1]
    ch = m // N_DEV
    hf = ch // 2

    def body(x_ref, w_ref, sx_ref, sw_ref, out_ref,
             gather_ref, comm_f, comm_b, stage_f, stage_b,
             send_f, recv_f, send_b, recv_b,
             ag_send_f, ag_recv_f, ag_send_b, ag_recv_b,
             credit_f, credit_b):
        my = lax.axis_index("i")
        left = lax.rem(my + N_DEV - 1, N_DEV)
        right = lax.rem(my + 1, N_DEV)

        barrier_sem = pltpu.get_barrier_semaphore()
        for nbr in (left, right):
            pl.semaphore_signal(barrier_sem, inc=1, device_id=(nbr,),
                                device_id_type=pl.DeviceIdType.MESH)
        pl.semaphore_wait(barrier_sem, 2)

        def frows(c):
            return pl.ds(c * ch, hf)

        def brows(c):
            return pl.ds(c * ch + hf, hf)

        def gemm_chunk(c):
            out_ref[pl.ds(c * ch, ch), :] = jnp.dot(
                x_ref[pl.ds(c * ch, ch), :], w_ref[...],
                preferred_element_type=jnp.float32)

        gemm_chunk(my)
        stage_f[...] = out_ref[frows(my), :].astype(jnp.bfloat16)
        stage_b[...] = out_ref[brows(my), :].astype(jnp.bfloat16)

        c_star_f = right
        c_star_b = left

        for s in range(N_DEV - 1):
            cr_f = lax.rem(my - s - 1 + N_DEV, N_DEV)
            cr_b = lax.rem(my + s + 1, N_DEV)
            if s > 0:
                pl.semaphore_wait(credit_f, 1)
                pl.semaphore_wait(credit_b, 1)
            rdma_f = pltpu.make_async_remote_copy(
                src_ref=stage_f,
                dst_ref=comm_f,
                send_sem=send_f,
                recv_sem=recv_f,
                device_id=(right,),
                device_id_type=pl.DeviceIdType.MESH,
            )
            rdma_b = pltpu.make_async_remote_copy(
                src_ref=stage_b,
                dst_ref=comm_b,
                send_sem=send_b,
                recv_sem=recv_b,
                device_id=(left,),
                device_id_type=pl.DeviceIdType.MESH,
            )
            rdma_f.start()
            rdma_b.start()
            if s == 0:
                for j in range(1, N_DEV):
                    gemm_chunk(lax.rem(my + j, N_DEV))
            rdma_f.wait()
            rdma_b.wait()
            acc_f = out_ref[frows(cr_f), :] + comm_f[...].astype(jnp.float32)
            acc_b = out_ref[brows(cr_b), :] + comm_b[...].astype(jnp.float32)
            if s < N_DEV - 2:
                stage_f[...] = acc_f.astype(jnp.bfloat16)
                stage_b[...] = acc_b.astype(jnp.bfloat16)
            else:
                gather_ref[frows(c_star_f), :] = acc_f.astype(jnp.bfloat16)
                gather_ref[brows(c_star_b), :] = acc_b.astype(jnp.bfloat16)
            pl.semaphore_signal(credit_f, inc=1, device_id=(left,),
                                device_id_type=pl.DeviceIdType.MESH)
            pl.semaphore_signal(credit_b, inc=1, device_id=(right,),
                                device_id_type=pl.DeviceIdType.MESH)
        pl.semaphore_wait(credit_f, 1)
        pl.semaphore_wait(credit_b, 1)

        scale = sx_ref[0] * sw_ref[0]

        def epilogue_half(rows_slice):
            out_ref[rows_slice, :] = jnp.maximum(
                gather_ref[rows_slice, :].astype(jnp.float32) * scale, 0.0)

        for s in range(N_DEV - 1):
            c_f = lax.rem(my + 1 - s + N_DEV, N_DEV)
            c_b = lax.rem(my - 1 + s + N_DEV, N_DEV)
            rdma_f = pltpu.make_async_remote_copy(
                src_ref=gather_ref.at[frows(c_f), :],
                dst_ref=gather_ref.at[frows(c_f), :],
                send_sem=ag_send_f,
                recv_sem=ag_recv_f.at[s],
                device_id=(right,),
                device_id_type=pl.DeviceIdType.MESH,
            )
            rdma_b = pltpu.make_async_remote_copy(
                src_ref=gather_ref.at[brows(c_b), :],
                dst_ref=gather_ref.at[brows(c_b), :],
                send_sem=ag_send_b,
                recv_sem=ag_recv_b.at[s],
                device_id=(left,),
                device_id_type=pl.DeviceIdType.MESH,
            )
            rdma_f.start()
            rdma_b.start()
            epilogue_half(frows(c_f))
            epilogue_half(brows(c_b))
            rdma_f.wait()
            rdma_b.wait()
        epilogue_half(frows(lax.rem(my + 2, N_DEV)))
        epilogue_half(brows(lax.rem(my - 2 + N_DEV, N_DEV)))

    out = pl.pallas_call(
        body,
        out_shape=jax.ShapeDtypeStruct((m, n), jnp.float32),
        in_specs=[
            pl.BlockSpec(memory_space=pltpu.VMEM),
            pl.BlockSpec(memory_space=pltpu.VMEM),
            pl.BlockSpec(memory_space=pltpu.SMEM),
            pl.BlockSpec(memory_space=pltpu.SMEM),
        ],
        out_specs=pl.BlockSpec(memory_space=pltpu.VMEM),
        scratch_shapes=[
            pltpu.VMEM((m, n), jnp.bfloat16),
            pltpu.VMEM((hf, n), jnp.bfloat16),
            pltpu.VMEM((hf, n), jnp.bfloat16),
            pltpu.VMEM((hf, n), jnp.bfloat16),
            pltpu.VMEM((hf, n), jnp.bfloat16),
            pltpu.SemaphoreType.DMA,
            pltpu.SemaphoreType.DMA,
            pltpu.SemaphoreType.DMA,
            pltpu.SemaphoreType.DMA,
            pltpu.SemaphoreType.DMA,
            pltpu.SemaphoreType.DMA((N_DEV - 1,)),
            pltpu.SemaphoreType.DMA,
            pltpu.SemaphoreType.DMA((N_DEV - 1,)),
            pltpu.SemaphoreType.REGULAR,
            pltpu.SemaphoreType.REGULAR,
        ],
        compiler_params=pltpu.CompilerParams(
            collective_id=0,
            vmem_limit_bytes=62 * 1024 * 1024,
        ),
    )(x.astype(jnp.bfloat16), w_mat.astype(jnp.bfloat16), scale_x, scale_w)
    return out


# device time: 215573 ns/iter; 3.3307x vs baseline; 1.1629x over previous
import jax
import jax.numpy as jnp
from jax import lax
from jax.experimental import pallas as pl
from jax.experimental.pallas import tpu as pltpu

N_DEV = 8
N_STR = 4


def kernel(x, w_mat, scale_x, scale_w):
    m, k_sh = x.shape
    n = w_mat.shape[1]
    ch = m // N_DEV
    sb = ch // N_STR

    def body(x_ref, w_ref, sx_ref, sw_ref, out_ref, gather_ref,
             c0, c1, c2, c3, t0, t1, t2, t3,
             rs_send, rs_recv, ag_send, ag_recv0, ag_recv1, ag_recv2,
             ag_recv3, credit):
        comm = [c0, c1, c2, c3]
        stage = [t0, t1, t2, t3]
        ag_recv = [ag_recv0, ag_recv1, ag_recv2, ag_recv3]

        my = lax.axis_index("i")
        left = lax.rem(my + N_DEV - 1, N_DEV)
        right = lax.rem(my + 1, N_DEV)

        dest = [right, right, left, left]
        csrc = [left, left, right, right]

        def rows(q, c):
            return pl.ds(c * ch + q * sb, sb)

        def rs_send_chunk(q, s):
            if q < 2:
                return lax.rem(my - s + N_DEV, N_DEV)
            return lax.rem(my + s, N_DEV)

        def rs_recv_chunk(q, s):
            if q < 2:
                return lax.rem(my - s - 1 + N_DEV, N_DEV)
            return lax.rem(my + s + 1, N_DEV)

        def ag_chunk(q, s):
            if q < 2:
                return lax.rem(my + 1 - s + N_DEV, N_DEV)
            return lax.rem(my - 1 + s + N_DEV, N_DEV)

        c_star = [right, right, left, left]

        barrier_sem = pltpu.get_barrier_semaphore()
        for nbr in (left, right):
            pl.semaphore_signal(barrier_sem, inc=1, device_id=(nbr,),
                                device_id_type=pl.DeviceIdType.MESH)
        pl.semaphore_wait(barrier_sem, 2)

        def gemm_chunk(c):
            out_ref[pl.ds(c * ch, ch), :] = jnp.dot(
                x_ref[pl.ds(c * ch, ch), :], w_ref[...],
                preferred_element_type=jnp.float32)

        def make_rs(q):
            return pltpu.make_async_remote_copy(
                src_ref=stage[q],
                dst_ref=comm[q],
                send_sem=rs_send.at[q],
                recv_sem=rs_recv.at[q],
                device_id=(dest[q],),
                device_id_type=pl.DeviceIdType.MESH,
            )

        def make_ag(q, s):
            c = ag_chunk(q, s)
            return pltpu.make_async_remote_copy(
                src_ref=gather_ref.at[rows(q, c), :],
                dst_ref=gather_ref.at[rows(q, c), :],
                send_sem=ag_send.at[q],
                recv_sem=ag_recv[q].at[s],
                device_id=(dest[q],),
                device_id_type=pl.DeviceIdType.MESH,
            )

        gemm_chunk(my)
        rs_d = [None] * N_STR
        for q in range(N_STR):
            stage[q][...] = out_ref[rows(q, my), :].astype(jnp.bfloat16)
        for q in range(N_STR):
            rs_d[q] = make_rs(q)
            rs_d[q].start()
        for j in range(1, N_DEV):
            gemm_chunk(lax.rem(my + j, N_DEV))

        ag_d = [None] * N_STR
        scale = sx_ref[0] * sw_ref[0]

        def epilogue(q, c):
            out_ref[rows(q, c), :] = jnp.maximum(
                gather_ref[rows(q, c), :].astype(jnp.float32) * scale, 0.0)

        for s in range(N_DEV - 1):
            for q in range(N_STR):
                rs_d[q].wait_recv()
                acc = (out_ref[rows(q, rs_recv_chunk(q, s)), :]
                       + comm[q][...].astype(jnp.float32))
                rs_d[q].wait_send()
                if s < N_DEV - 2:
                    stage[q][...] = acc.astype(jnp.bfloat16)
                else:
                    gather_ref[rows(q, c_star[q]), :] = acc.astype(jnp.bfloat16)
                pl.semaphore_signal(credit.at[q], inc=1,
                                    device_id=(csrc[q],),
                                    device_id_type=pl.DeviceIdType.MESH)
                if s < N_DEV - 2:
                    pl.semaphore_wait(credit.at[q], 1)
                    rs_d[q] = make_rs(q)
                    rs_d[q].start()
                else:
                    ag_d[q] = make_ag(q, 0)
                    ag_d[q].start()
                    epilogue(q, c_star[q])
        for q in range(N_STR):
            pl.semaphore_wait(credit.at[q], 1)

        for s in range(N_DEV - 1):
            for q in range(N_STR):
                ag_d[q].wait_recv()
                ag_d[q].wait_send()
                c = ag_chunk(q, s + 1)
                if s < N_DEV - 2:
                    ag_d[q] = make_ag(q, s + 1)
                    ag_d[q].start()
                epilogue(q, c)

    out = pl.pallas_call(
        body,
        out_shape=jax.ShapeDtypeStruct((m, n), jnp.float32),
        in_specs=[
            pl.BlockSpec(memory_space=pltpu.VMEM),
            pl.BlockSpec(memory_space=pltpu.VMEM),
            pl.BlockSpec(memory_space=pltpu.SMEM),
            pl.BlockSpec(memory_space=pltpu.SMEM),
        ],
        out_specs=pl.BlockSpec(memory_space=pltpu.VMEM),
        scratch_shapes=(
            [pltpu.VMEM((m, n), jnp.bfloat16)]
            + [pltpu.VMEM((sb, n), jnp.bfloat16)] * 4
            + [pltpu.VMEM((sb, n), jnp.bfloat16)] * 4
            + [
                pltpu.SemaphoreType.DMA((N_STR,)),
                pltpu.SemaphoreType.DMA((N_STR,)),
                pltpu.SemaphoreType.DMA((N_STR,)),
                pltpu.SemaphoreType.DMA((N_DEV - 1,)),
                pltpu.SemaphoreType.DMA((N_DEV - 1,)),
                pltpu.SemaphoreType.DMA((N_DEV - 1,)),
                pltpu.SemaphoreType.DMA((N_DEV - 1,)),
                pltpu.SemaphoreType.REGULAR((N_STR,)),
            ]
        ),
        compiler_params=pltpu.CompilerParams(
            collective_id=0,
            vmem_limit_bytes=62 * 1024 * 1024,
        ),
    )(x.astype(jnp.bfloat16), w_mat.astype(jnp.bfloat16), scale_x, scale_w)
    return out


# device time: 212244 ns/iter; 3.3830x vs baseline; 1.0157x over previous
import jax
import jax.numpy as jnp
from jax import lax
from jax.experimental import pallas as pl
from jax.experimental.pallas import tpu as pltpu

N_DEV = 8
N_STR = 4
STREAM_ORDER = (0, 2, 1, 3)


def kernel(x, w_mat, scale_x, scale_w):
    m, k_sh = x.shape
    n = w_mat.shape[1]
    ch = m // N_DEV
    sb = ch // N_STR

    def body(x_ref, w_ref, sx_ref, sw_ref, out_ref, gather_ref,
             c0, c1, c2, c3, t0, t1, t2, t3,
             rs_send, rs_recv, ag_send, ag_recv0, ag_recv1, ag_recv2,
             ag_recv3, credit):
        comm = [c0, c1, c2, c3]
        stage = [t0, t1, t2, t3]
        ag_recv = [ag_recv0, ag_recv1, ag_recv2, ag_recv3]

        my = lax.axis_index("i")
        left = lax.rem(my + N_DEV - 1, N_DEV)
        right = lax.rem(my + 1, N_DEV)

        dest = [right, right, left, left]
        csrc = [left, left, right, right]

        def rows(q, c):
            return pl.ds(c * ch + q * sb, sb)

        def rs_send_chunk(q, s):
            if q < 2:
                return lax.rem(my - s + N_DEV, N_DEV)
            return lax.rem(my + s, N_DEV)

        def rs_recv_chunk(q, s):
            if q < 2:
                return lax.rem(my - s - 1 + N_DEV, N_DEV)
            return lax.rem(my + s + 1, N_DEV)

        def ag_chunk(q, s):
            if q < 2:
                return lax.rem(my + 1 - s + N_DEV, N_DEV)
            return lax.rem(my - 1 + s + N_DEV, N_DEV)

        c_star = [right, right, left, left]

        barrier_sem = pltpu.get_barrier_semaphore()
        for nbr in (left, right):
            pl.semaphore_signal(barrier_sem, inc=1, device_id=(nbr,),
                                device_id_type=pl.DeviceIdType.MESH)
        pl.semaphore_wait(barrier_sem, 2)

        def gemm_chunk(c):
            out_ref[pl.ds(c * ch, ch), :] = jnp.dot(
                x_ref[pl.ds(c * ch, ch), :], w_ref[...],
                preferred_element_type=jnp.float32)

        def make_rs(q):
            return pltpu.make_async_remote_copy(
                src_ref=stage[q],
                dst_ref=comm[q],
                send_sem=rs_send.at[q],
                recv_sem=rs_recv.at[q],
                device_id=(dest[q],),
                device_id_type=pl.DeviceIdType.MESH,
            )

        def make_ag(q, s):
            c = ag_chunk(q, s)
            return pltpu.make_async_remote_copy(
                src_ref=gather_ref.at[rows(q, c), :],
                dst_ref=gather_ref.at[rows(q, c), :],
                send_sem=ag_send.at[q],
                recv_sem=ag_recv[q].at[s],
                device_id=(dest[q],),
                device_id_type=pl.DeviceIdType.MESH,
            )

        gemm_chunk(my)
        rs_d = [None] * N_STR
        for q in range(N_STR):
            stage[q][...] = out_ref[rows(q, my), :].astype(jnp.bfloat16)
        for q in STREAM_ORDER:
            rs_d[q] = make_rs(q)
            rs_d[q].start()
        for j in range(1, N_DEV):
            gemm_chunk(lax.rem(my + j, N_DEV))

        ag_d = [None] * N_STR
        scale = sx_ref[0] * sw_ref[0]

        def epilogue(q, c):
            out_ref[rows(q, c), :] = jnp.maximum(
                gather_ref[rows(q, c), :].astype(jnp.float32) * scale, 0.0)

        for s in range(N_DEV - 1):
            for q in STREAM_ORDER:
                rs_d[q].wait_recv()
                acc = (out_ref[rows(q, rs_recv_chunk(q, s)), :]
                       + comm[q][...].astype(jnp.float32))
                rs_d[q].wait_send()
                if s < N_DEV - 2:
                    stage[q][...] = acc.astype(jnp.bfloat16)
                else:
                    gather_ref[rows(q, c_star[q]), :] = acc.astype(jnp.bfloat16)
                pl.semaphore_signal(credit.at[q], inc=1,
                                    device_id=(csrc[q],),
                                    device_id_type=pl.DeviceIdType.MESH)
                if s < N_DEV - 2:
                    pl.semaphore_wait(credit.at[q], 1)
                    rs_d[q] = make_rs(q)
                    rs_d[q].start()
                else:
                    ag_d[q] = make_ag(q, 0)
                    ag_d[q].start()
                    epilogue(q, c_star[q])
        for q in range(N_STR):
            pl.semaphore_wait(credit.at[q], 1)

        for s in range(N_DEV - 1):
            for q in STREAM_ORDER:
                ag_d[q].wait_recv()
                ag_d[q].wait_send()
                c = ag_chunk(q, s + 1)
                if s < N_DEV - 2:
                    ag_d[q] = make_ag(q, s + 1)
                    ag_d[q].start()
                epilogue(q, c)

    out = pl.pallas_call(
        body,
        out_shape=jax.ShapeDtypeStruct((m, n), jnp.float32),
        in_specs=[
            pl.BlockSpec(memory_space=pltpu.VMEM),
            pl.BlockSpec(memory_space=pltpu.VMEM),
            pl.BlockSpec(memory_space=pltpu.SMEM),
            pl.BlockSpec(memory_space=pltpu.SMEM),
        ],
        out_specs=pl.BlockSpec(memory_space=pltpu.VMEM),
        scratch_shapes=(
            [pltpu.VMEM((m, n), jnp.bfloat16)]
            + [pltpu.VMEM((sb, n), jnp.bfloat16)] * 4
            + [pltpu.VMEM((sb, n), jnp.bfloat16)] * 4
            + [
                pltpu.SemaphoreType.DMA((N_STR,)),
                pltpu.SemaphoreType.DMA((N_STR,)),
                pltpu.SemaphoreType.DMA((N_STR,)),
                pltpu.SemaphoreType.DMA((N_DEV - 1,)),
                pltpu.SemaphoreType.DMA((N_DEV - 1,)),
                pltpu.SemaphoreType.DMA((N_DEV - 1,)),
                pltpu.SemaphoreType.DMA((N_DEV - 1,)),
                pltpu.SemaphoreType.REGULAR((N_STR,)),
            ]
        ),
        compiler_params=pltpu.CompilerParams(
            collective_id=0,
            vmem_limit_bytes=62 * 1024 * 1024,
        ),
    )(x.astype(jnp.bfloat16), w_mat.astype(jnp.bfloat16), scale_x, scale_w)
    return out


# device time: 211964 ns/iter; 3.3874x vs baseline; 1.0013x over previous
import jax
import jax.numpy as jnp
from jax import lax
from jax.experimental import pallas as pl
from jax.experimental.pallas import tpu as pltpu

N_DEV = 8
N_STR = 4
STREAM_ORDER = (0, 2, 1, 3)


def kernel(x, w_mat, scale_x, scale_w):
    m, k_sh = x.shape
    n = w_mat.shape[1]
    ch = m // N_DEV
    sb = ch // N_STR

    def body(x_ref, w_ref, sx_ref, sw_ref, out_ref, gather_ref,
             c0, c1, c2, c3, t0, t1, t2, t3,
             rs_send, rs_recv, ag_send, ag_recv0, ag_recv1, ag_recv2,
             ag_recv3, credit):
        comm = [c0, c1, c2, c3]
        stage = [t0, t1, t2, t3]
        ag_recv = [ag_recv0, ag_recv1, ag_recv2, ag_recv3]

        my = lax.axis_index("i")
        left = lax.rem(my + N_DEV - 1, N_DEV)
        right = lax.rem(my + 1, N_DEV)

        dest = [right, right, left, left]
        csrc = [left, left, right, right]

        def rows(q, c):
            return pl.ds(c * ch + q * sb, sb)

        def rs_send_chunk(q, s):
            if q < 2:
                return lax.rem(my - s + N_DEV, N_DEV)
            return lax.rem(my + s, N_DEV)

        def rs_recv_chunk(q, s):
            if q < 2:
                return lax.rem(my - s - 1 + N_DEV, N_DEV)
            return lax.rem(my + s + 1, N_DEV)

        def ag_chunk(q, s):
            if q < 2:
                return lax.rem(my + 1 - s + N_DEV, N_DEV)
            return lax.rem(my - 1 + s + N_DEV, N_DEV)

        c_star = [right, right, left, left]

        barrier_sem = pltpu.get_barrier_semaphore()
        for nbr in (left, right):
            pl.semaphore_signal(barrier_sem, inc=1, device_id=(nbr,),
                                device_id_type=pl.DeviceIdType.MESH)
        pl.semaphore_wait(barrier_sem, 2)

        def gemm_chunk(c):
            out_ref[pl.ds(c * ch, ch), :] = jnp.dot(
                x_ref[pl.ds(c * ch, ch), :], w_ref[...],
                preferred_element_type=jnp.float32)

        def make_rs(q, s):
            return pltpu.make_async_remote_copy(
                src_ref=stage[q],
                dst_ref=comm[q].at[s % 2],
                send_sem=rs_send.at[q],
                recv_sem=rs_recv.at[q, s % 2],
                device_id=(dest[q],),
                device_id_type=pl.DeviceIdType.MESH,
            )

        def make_ag(q, s):
            c = ag_chunk(q, s)
            return pltpu.make_async_remote_copy(
                src_ref=gather_ref.at[rows(q, c), :],
                dst_ref=gather_ref.at[rows(q, c), :],
                send_sem=ag_send.at[q],
                recv_sem=ag_recv[q].at[s],
                device_id=(dest[q],),
                device_id_type=pl.DeviceIdType.MESH,
            )

        gemm_chunk(my)
        rs_d = [None] * N_STR
        for q in range(N_STR):
            stage[q][...] = out_ref[rows(q, my), :].astype(jnp.bfloat16)
        for q in STREAM_ORDER:
            rs_d[q] = make_rs(q, 0)
            rs_d[q].start()
        for j in range(1, N_DEV):
            gemm_chunk(lax.rem(my + j, N_DEV))

        ag_d = [None] * N_STR
        scale = sx_ref[0] * sw_ref[0]

        def epilogue(q, c):
            out_ref[rows(q, c), :] = jnp.maximum(
                gather_ref[rows(q, c), :].astype(jnp.float32) * scale, 0.0)

        for s in range(N_DEV - 1):
            for q in STREAM_ORDER:
                rs_d[q].wait_recv()
                acc = (out_ref[rows(q, rs_recv_chunk(q, s)), :]
                       + comm[q][s % 2].astype(jnp.float32))
                rs_d[q].wait_send()
                if s < N_DEV - 2:
                    stage[q][...] = acc.astype(jnp.bfloat16)
                else:
                    gather_ref[rows(q, c_star[q]), :] = acc.astype(jnp.bfloat16)
                pl.semaphore_signal(credit.at[q], inc=1,
                                    device_id=(csrc[q],),
                                    device_id_type=pl.DeviceIdType.MESH)
                if s < N_DEV - 2:
                    if s >= 1:
                        pl.semaphore_wait(credit.at[q], 1)
                    rs_d[q] = make_rs(q, s + 1)
                    rs_d[q].start()
                else:
                    ag_d[q] = make_ag(q, 0)
                    ag_d[q].start()
                    epilogue(q, c_star[q])
        for q in range(N_STR):
            pl.semaphore_wait(credit.at[q], 2)

        for s in range(N_DEV - 1):
            for q in STREAM_ORDER:
                ag_d[q].wait_recv()
                ag_d[q].wait_send()
                c = ag_chunk(q, s + 1)
                if s < N_DEV - 2:
                    ag_d[q] = make_ag(q, s + 1)
                    ag_d[q].start()
                epilogue(q, c)

    out = pl.pallas_call(
        body,
        out_shape=jax.ShapeDtypeStruct((m, n), jnp.float32),
        in_specs=[
            pl.BlockSpec(memory_space=pltpu.VMEM),
            pl.BlockSpec(memory_space=pltpu.VMEM),
            pl.BlockSpec(memory_space=pltpu.SMEM),
            pl.BlockSpec(memory_space=pltpu.SMEM),
        ],
        out_specs=pl.BlockSpec(memory_space=pltpu.VMEM),
        scratch_shapes=(
            [pltpu.VMEM((m, n), jnp.bfloat16)]
            + [pltpu.VMEM((2, sb, n), jnp.bfloat16)] * 4
            + [pltpu.VMEM((sb, n), jnp.bfloat16)] * 4
            + [
                pltpu.SemaphoreType.DMA((N_STR,)),
                pltpu.SemaphoreType.DMA((N_STR, 2)),
                pltpu.SemaphoreType.DMA((N_STR,)),
                pltpu.SemaphoreType.DMA((N_DEV - 1,)),
                pltpu.SemaphoreType.DMA((N_DEV - 1,)),
                pltpu.SemaphoreType.DMA((N_DEV - 1,)),
                pltpu.SemaphoreType.DMA((N_DEV - 1,)),
                pltpu.SemaphoreType.REGULAR((N_STR,)),
            ]
        ),
        compiler_params=pltpu.CompilerParams(
            collective_id=0,
            vmem_limit_bytes=62 * 1024 * 1024,
        ),
    )(x.astype(jnp.bfloat16), w_mat.astype(jnp.bfloat16), scale_x, scale_w)
    return out
